# baseline (speedup 1.0000x reference)
import dataclasses
import functools

import jax
import jax.numpy as jnp
import numpy as np
from jax import lax
from jax.experimental import pallas as pl
from jax.experimental.pallas import tpu as pltpu
from jax.experimental.pallas import tpu_sc as plsc

_F = 26
_D = 16
_A = 16
_G = 8
_L = 128
_CHUNK = 128
_NW = 32
_CROWS = 40


def _sc_params():
    cp = pltpu.CompilerParams()
    for fld, val in (("needs_layout_passes", False),
                     ("use_tc_tiling_on_sc", True)):
        if fld in pltpu.CompilerParams.__dataclass_fields__:
            cp = dataclasses.replace(cp, **{fld: val})
    return cp


def _sc_depad(emb_table):
    nrows = emb_table.shape[0]
    n_chunks = nrows // (_CROWS * _G)
    mesh = plsc.VectorSubcoreMesh(core_axis_name="core", subcore_axis_name="subcore")
    iters = -(-n_chunks // _NW)

    @functools.partial(
        pl.kernel,
        compiler_params=_sc_params(),
        out_type=jax.ShapeDtypeStruct((nrows // _G, _L), jnp.float32),
        mesh=mesh,
        scratch_types=[
            pltpu.VMEM((2, _CROWS * _G, _D), jnp.float32),
            pltpu.VMEM((2, _CROWS, _L), jnp.float32),
            pltpu.SemaphoreType.DMA,
            pltpu.SemaphoreType.DMA,
        ],
    )
    def depad_kernel(emb_hbm, out_hbm, in_v, st_v, sem_i, sem_o):
        wid = lax.axis_index("subcore") * 2 + lax.axis_index("core")

        def in_copy(k, slot):
            c = wid + _NW * k
            return pltpu.make_async_copy(
                emb_hbm.at[pl.ds(c * _CROWS * _G, _CROWS * _G), :],
                in_v.at[slot], sem_i)

        def out_copy(k, slot):
            c = wid + _NW * k
            return pltpu.make_async_copy(
                st_v.at[slot], out_hbm.at[pl.ds(c * _CROWS, _CROWS), :], sem_o)

        def valid(k):
            return wid + _NW * k < n_chunks

        @pl.when(valid(0))
        def _():
            in_copy(0, 0).start()

        @pl.loop(0, iters)
        def _(k):
            slot = k % 2

            @pl.when(valid(k + 1))
            def _():
                in_copy(k + 1, 1 - slot).start()

            @pl.when(valid(k))
            def _():
                in_copy(k, slot).wait()

                @pl.loop(0, _CROWS * _G // 16)
                def _(j16):
                    for u in range(16):
                        st_v[slot, j16 * 2 + u // _G,
                             pl.ds((u % _G) * _D, _D)] = (
                            in_v[slot, j16 * 16 + u, :])

                @pl.when(k >= 2)
                def _():
                    out_copy(k - 2, slot).wait()

                out_copy(k, slot).start()

        k_last = (n_chunks - 1 - wid) // _NW

        @pl.when(k_last >= 1)
        def _():
            out_copy(k_last - 1, (k_last - 1) % 2).wait()
        out_copy(k_last, k_last % 2).wait()

    return depad_kernel(emb_table)


def _sc_gather(emb128, lin128, erow, esub, lrow, lcol, batch):
    mesh = plsc.VectorSubcoreMesh(core_axis_name="core", subcore_axis_name="subcore")

    @functools.partial(
        pl.kernel,
        compiler_params=_sc_params(),
        out_type=(
            jax.ShapeDtypeStruct((batch // _G, _F, _G * _G), jnp.float32),
            jax.ShapeDtypeStruct((batch,), jnp.float32),
        ),
        mesh=mesh,
        scratch_types=[
            pltpu.VMEM((_CHUNK,), jnp.int32),
            pltpu.VMEM((_CHUNK,), jnp.int32),
            pltpu.VMEM((_CHUNK, _L), jnp.float32),
            pltpu.VMEM((_CHUNK // _G, 1, _G * _G), jnp.float32),
            pltpu.VMEM((_CHUNK,), jnp.int32),
            pltpu.VMEM((_CHUNK,), jnp.int32),
            pltpu.VMEM((_CHUNK, _L), jnp.float32),
            pltpu.VMEM((_CHUNK,), jnp.float32),
            pltpu.SemaphoreType.DMA,
            pltpu.SemaphoreType.DMA,
        ],
    )
    def gather_kernel(emb_hbm, lin_hbm, er_hbm, es_hbm, lr_hbm, lc_hbm,
                      oe_hbm, ol_hbm,
                      eidx_v, esub_v, erows_v, pack_v, lrow_v, lcol_v,
                      lrows_v, acc_v, sem_e, sem_l):
        wid = lax.axis_index("subcore") * 2 + lax.axis_index("core")
        b0 = wid * _CHUNK
        zeros16 = jnp.zeros((16,), jnp.float32)
        iota16 = lax.iota(jnp.int32, 16)

        @pl.loop(0, _CHUNK // 16)
        def _(c):
            acc_v[pl.ds(c * 16, 16)] = zeros16

        @pl.loop(0, _F)
        def _(f):
            base = f * batch + b0
            pltpu.sync_copy(er_hbm.at[pl.ds(base, _CHUNK)], eidx_v)
            pltpu.sync_copy(es_hbm.at[pl.ds(base, _CHUNK)], esub_v)
            pltpu.async_copy(emb_hbm.at[eidx_v], erows_v, sem_e).wait()

            half = iota16 // _G
            lane8 = iota16 % _G

            @pl.loop(0, _CHUNK // 2)
            def _(c):
                rows = jnp.full((16,), 2 * c, jnp.int32) + half
                s16 = plsc.load_gather(esub_v, [rows])
                cols = s16 * _G + lane8
                vals = plsc.load_gather(erows_v, [rows, cols])
                r = 2 * c
                pack_v[r // _G, 0, pl.ds((r % _G) * _G, 16)] = vals

            pltpu.sync_copy(
                pack_v, oe_hbm.at[pl.ds(wid * (_CHUNK // _G), _CHUNK // _G),
                                  pl.ds(f, 1), :])
            pltpu.sync_copy(lr_hbm.at[pl.ds(base, _CHUNK)], lrow_v)
            pltpu.sync_copy(lc_hbm.at[pl.ds(base, _CHUNK)], lcol_v)
            pltpu.async_copy(lin_hbm.at[lrow_v], lrows_v, sem_l).wait()

            @pl.loop(0, _CHUNK // 16)
            def _(c):
                rows = iota16 + c * 16
                cols = lcol_v[pl.ds(c * 16, 16)]
                vals = plsc.load_gather(lrows_v, [rows, cols])
                acc_v[pl.ds(c * 16, 16)] += vals

        pltpu.sync_copy(acc_v, ol_hbm.at[pl.ds(b0, _CHUNK)])

    return gather_kernel(emb128, lin128, erow, esub, lrow, lcol)


def _afm_body(e_ref, lin_ref, wlo_ref, whi_ref, bdp_ref, flo_ref, fhi_ref,
              attn_bt_ref, bias_ref, out_ref):
    n_groups = e_ref.shape[0]
    pair_iota = jax.lax.broadcasted_iota(jnp.int32, (_F * _F, 1), 0)
    valid = (pair_iota // _F) < (pair_iota % _F)
    attn_bt = attn_bt_ref[...]
    rows_lo, rows_hi = [], []
    for g in range(n_groups):
        e8i = jax.lax.bitcast_convert_type(
            e_ref[g, :, :], jnp.int32)
        e_lo = jax.lax.bitcast_convert_type(
            e8i << 16, jnp.float32)
        e_hi = jax.lax.bitcast_convert_type(
            e8i & jnp.int32(-65536), jnp.float32)
        i_lo = (e_lo[:, None, :] * e_lo[None, :, :]).reshape(_F * _F, 64)
        i_hi = (e_hi[:, None, :] * e_hi[None, :, :]).reshape(_F * _F, 64)
        attn = jnp.maximum(
            jnp.dot(i_lo, wlo_ref[...], preferred_element_type=jnp.float32)
            + jnp.dot(i_hi, whi_ref[...], preferred_element_type=jnp.float32)
            + attn_bt, 0.0)
        logits = jnp.dot(attn, bdp_ref[...],
                         preferred_element_type=jnp.float32)
        logits = jnp.where(valid, logits, -1e30)
        mx = jnp.max(logits, axis=0, keepdims=True)
        ex = jnp.where(valid, jnp.exp(logits - mx), 0.0)
        scores = ex / jnp.sum(ex, axis=0, keepdims=True)
        rows_lo.append(jnp.sum(scores * i_lo, axis=0, keepdims=True))
        rows_hi.append(jnp.sum(scores * i_hi, axis=0, keepdims=True))
    ao_lo = jnp.concatenate(rows_lo, axis=0)
    ao_hi = jnp.concatenate(rows_hi, axis=0)
    afm = (jnp.dot(ao_lo, flo_ref[...], preferred_element_type=jnp.float32)
           + jnp.dot(ao_hi, fhi_ref[...], preferred_element_type=jnp.float32))
    out_ref[...] = afm + lin_ref[...] + bias_ref[...]


def _afm_tc(e_packed, lin_sums8, wlo, whi, bdp, flo, fhi, attn_bt, bias,
            batch):
    n_groups_blk = _CHUNK // _G
    grid = (batch // _CHUNK,)
    return pl.pallas_call(
        _afm_body,
        grid=grid,
        in_specs=[
            pl.BlockSpec((n_groups_blk, _F, _G * _G), lambda i: (i, 0, 0)),
            pl.BlockSpec((n_groups_blk, _G), lambda i: (i, 0)),
            pl.BlockSpec((64, 128), lambda i: (0, 0)),
            pl.BlockSpec((64, 128), lambda i: (0, 0)),
            pl.BlockSpec((128, 64), lambda i: (0, 0)),
            pl.BlockSpec((64, _G), lambda i: (0, 0)),
            pl.BlockSpec((64, _G), lambda i: (0, 0)),
            pl.BlockSpec((1, _G * _D), lambda i: (0, 0)),
            pl.BlockSpec((1, 1), lambda i: (0, 0)),
        ],
        out_specs=pl.BlockSpec((n_groups_blk, _G), lambda i: (i, 0)),
        out_shape=jax.ShapeDtypeStruct((batch // _G, _G), jnp.float32),
        compiler_params=pltpu.CompilerParams(
            dimension_semantics=("parallel",)),
    )(e_packed, lin_sums8, wlo, whi, bdp, flo, fhi, attn_bt, bias)


def kernel(x, emb_table, lin_table, lin_bias, attn_W, attn_b, proj_W, proj_b,
           fc_W, fc_b):
    batch, num_fields = x.shape
    field_dim = emb_table.shape[0] // num_fields
    offsets = (jnp.arange(num_fields, dtype=x.dtype) * field_dim)[None, :]
    idxf = ((x + offsets).T).reshape(-1)

    emb_bf = emb_table.astype(jnp.bfloat16)
    emb128 = jax.lax.bitcast_convert_type(jax.lax.bitcast_convert_type(
        emb_bf.reshape(-1, _L, 2), jnp.int32), jnp.float32)

    nlin = lin_table.shape[0]
    npad = (-nlin) % _L
    lin128 = jnp.pad(lin_table.reshape(-1), (0, npad)).reshape(-1, _L)
    rows_per = 16

    e_packed, lin_sums = _sc_gather(emb128, lin128,
                                    idxf // rows_per, idxf % rows_per,
                                    idxf // _L, idxf % _L, batch)

    eye = jnp.eye(_G, dtype=jnp.float32)
    wlo = jnp.kron(eye, attn_W[0::2, :])
    whi = jnp.kron(eye, attn_W[1::2, :])
    bdp = jnp.kron(eye, jnp.outer(proj_W[:, 0], jnp.ones((_G,), jnp.float32)))
    flo = jnp.kron(eye, fc_W[0::2])
    fhi = jnp.kron(eye, fc_W[1::2])
    attn_bt = jnp.tile(attn_b, (_G,))[None, :]
    bias = (fc_b + lin_bias).reshape(1, 1)

    out = _afm_tc(e_packed, lin_sums.reshape(batch // _G, _G), wlo, whi, bdp,
                  flo, fhi, attn_bt, bias, batch)
    return out.reshape(batch)


# speedup vs baseline: 7.0259x; 7.0259x over previous
import dataclasses
import functools

import jax
import jax.numpy as jnp
import numpy as np
from jax import lax
from jax.experimental import pallas as pl
from jax.experimental.pallas import tpu as pltpu
from jax.experimental.pallas import tpu_sc as plsc

_F = 26
_D = 16
_A = 16
_G = 8
_L = 128
_CHUNK = 128


def _sc_gather(emb_table, lin128, eidx, lrow, lcol, batch):
    mesh = plsc.VectorSubcoreMesh(core_axis_name="core", subcore_axis_name="subcore")
    cp = pltpu.CompilerParams()
    for fld, val in (("needs_layout_passes", False),
                     ("use_tc_tiling_on_sc", False)):
        if fld in pltpu.CompilerParams.__dataclass_fields__:
            cp = dataclasses.replace(cp, **{fld: val})

    @functools.partial(
        pl.kernel,
        compiler_params=cp,
        out_type=(
            jax.ShapeDtypeStruct((batch // _G, _F, _G * _D), jnp.float32),
            jax.ShapeDtypeStruct((batch,), jnp.float32),
        ),
        mesh=mesh,
        scratch_types=[
            pltpu.VMEM((_CHUNK,), jnp.int32),
            pltpu.VMEM((_CHUNK, _D), jnp.float32),
            pltpu.VMEM((_CHUNK // _G, 1, _G * _D), jnp.float32),
            pltpu.VMEM((_CHUNK,), jnp.int32),
            pltpu.VMEM((_CHUNK,), jnp.int32),
            pltpu.VMEM((_CHUNK, _L), jnp.float32),
            pltpu.VMEM((_CHUNK,), jnp.float32),
            pltpu.SemaphoreType.DMA,
            pltpu.SemaphoreType.DMA,
        ],
    )
    def gather_kernel(emb_hbm, lin_hbm, ei_hbm, lr_hbm, lc_hbm,
                      oe_hbm, ol_hbm,
                      eidx_v, erows_v, pack_v, lrow_v, lcol_v,
                      lrows_v, acc_v, sem_e, sem_l):
        wid = lax.axis_index("subcore") * 2 + lax.axis_index("core")
        b0 = wid * _CHUNK
        zeros16 = jnp.zeros((16,), jnp.float32)
        iota16 = lax.iota(jnp.int32, 16)

        @pl.loop(0, _CHUNK // 16)
        def _(c):
            acc_v[pl.ds(c * 16, 16)] = zeros16

        @pl.loop(0, _F)
        def _(f):
            base = f * batch + b0
            pltpu.sync_copy(ei_hbm.at[pl.ds(base, _CHUNK)], eidx_v)
            pltpu.async_copy(emb_hbm.at[eidx_v], erows_v, sem_e).wait()

            @pl.loop(0, _CHUNK)
            def _(r):
                pack_v[r // _G, 0, pl.ds((r % _G) * _D, _D)] = erows_v[r, :]

            pltpu.sync_copy(
                pack_v, oe_hbm.at[pl.ds(wid * (_CHUNK // _G), _CHUNK // _G),
                                  pl.ds(f, 1), :])
            pltpu.sync_copy(lr_hbm.at[pl.ds(base, _CHUNK)], lrow_v)
            pltpu.sync_copy(lc_hbm.at[pl.ds(base, _CHUNK)], lcol_v)
            pltpu.async_copy(lin_hbm.at[lrow_v], lrows_v, sem_l).wait()

            @pl.loop(0, _CHUNK // 16)
            def _(c):
                rows = iota16 + c * 16
                cols = lcol_v[pl.ds(c * 16, 16)]
                vals = plsc.load_gather(lrows_v, [rows, cols])
                acc_v[pl.ds(c * 16, 16)] += vals

        pltpu.sync_copy(acc_v, ol_hbm.at[pl.ds(b0, _CHUNK)])

    return gather_kernel(emb_table, lin128, eidx, lrow, lcol)


def _afm_body(e_ref, lin_ref, bdw_ref, bdp_ref, bdf_ref, attn_bt_ref, bias_ref,
              out_ref):
    n_groups = e_ref.shape[0]
    pair_iota = jax.lax.broadcasted_iota(jnp.int32, (_F * _F, 1), 0)
    valid = (pair_iota // _F) < (pair_iota % _F)
    attn_bt = attn_bt_ref[...]
    rows = []
    for g in range(n_groups):
        e8 = e_ref[g, :, :]
        i8 = (e8[:, None, :] * e8[None, :, :]).reshape(_F * _F, _G * _D)
        attn = jnp.maximum(
            jnp.dot(i8, bdw_ref[...], preferred_element_type=jnp.float32)
            + attn_bt, 0.0)
        logits = jnp.dot(attn, bdp_ref[...],
                         preferred_element_type=jnp.float32)
        logits = jnp.where(valid, logits, -1e30)
        mx = jnp.max(logits, axis=0, keepdims=True)
        ex = jnp.where(valid, jnp.exp(logits - mx), 0.0)
        scores = ex / jnp.sum(ex, axis=0, keepdims=True)
        rows.append(jnp.sum(scores * i8, axis=0, keepdims=True))
    ao = jnp.concatenate(rows, axis=0)
    afm = jnp.dot(ao, bdf_ref[...], preferred_element_type=jnp.float32)
    out_ref[...] = afm + lin_ref[...] + bias_ref[...]


def _afm_tc(e_packed, lin_sums8, bdw, bdp, bdf, attn_bt, bias, batch):
    n_groups_blk = _CHUNK // _G
    grid = (batch // _CHUNK,)
    return pl.pallas_call(
        _afm_body,
        grid=grid,
        in_specs=[
            pl.BlockSpec((n_groups_blk, _F, _G * _D), lambda i: (i, 0, 0)),
            pl.BlockSpec((n_groups_blk, _G), lambda i: (i, 0)),
            pl.BlockSpec((_G * _D, _G * _D), lambda i: (0, 0)),
            pl.BlockSpec((_G * _D, _G * _D), lambda i: (0, 0)),
            pl.BlockSpec((_G * _D, _G), lambda i: (0, 0)),
            pl.BlockSpec((1, _G * _D), lambda i: (0, 0)),
            pl.BlockSpec((1, 1), lambda i: (0, 0)),
        ],
        out_specs=pl.BlockSpec((n_groups_blk, _G), lambda i: (i, 0)),
        out_shape=jax.ShapeDtypeStruct((batch // _G, _G), jnp.float32),
        compiler_params=pltpu.CompilerParams(
            dimension_semantics=("parallel",)),
    )(e_packed, lin_sums8, bdw, bdp, bdf, attn_bt, bias)


def kernel(x, emb_table, lin_table, lin_bias, attn_W, attn_b, proj_W, proj_b,
           fc_W, fc_b):
    batch, num_fields = x.shape
    field_dim = emb_table.shape[0] // num_fields
    offsets = (jnp.arange(num_fields, dtype=x.dtype) * field_dim)[None, :]
    idxf = ((x + offsets).T).reshape(-1)

    nlin = lin_table.shape[0]
    npad = (-nlin) % _L
    lin128 = jnp.pad(lin_table.reshape(-1), (0, npad)).reshape(-1, _L)

    e_packed, lin_sums = _sc_gather(emb_table, lin128, idxf,
                                    idxf // _L, idxf % _L, batch)

    eye = jnp.eye(_G, dtype=jnp.float32)
    bdw = jnp.kron(eye, attn_W)
    bdp = jnp.kron(eye, jnp.outer(proj_W[:, 0], jnp.ones((_D,), jnp.float32)))
    bdf = jnp.kron(eye, fc_W)
    attn_bt = jnp.tile(attn_b, (_G,))[None, :]
    bias = (fc_b + lin_bias).reshape(1, 1)

    out = _afm_tc(e_packed, lin_sums.reshape(batch // _G, _G), bdw, bdp, bdf,
                  attn_bt, bias, batch)
    return out.reshape(batch)
